# baseline (speedup 1.0000x reference)
import functools

import jax
import jax.numpy as jnp
from jax.experimental import pallas as pl
from jax.experimental.pallas import tpu as pltpu
from jax.experimental.pallas import tpu_sc as plsc

NUM_CLASSES = 117
NUM_OBJ = 80
N_DET = 1000
N_HUM = 16
POOL = 4
NPTS = POOL * POOL
C = 64
FH = FW = 64
THRESH = 0.2
P = N_HUM * N_DET
NIDX = P * NPTS
REP = 128
GATHER_WIN = 256


def _idx_body(boxes_ref, out_ref):
    g8 = pl.program_id(0)
    h = pl.program_id(1)
    b = boxes_ref[...]
    rowi = jax.lax.broadcasted_iota(jnp.int32, (N_DET, 1), 0)
    hm = rowi == h
    bh = jnp.sum(jnp.where(hm, b, 0.0), axis=0, keepdims=True)
    x1h, y1h, x2h, y2h = (bh[:, 0:1], bh[:, 1:2], bh[:, 2:3], bh[:, 3:4])
    x1o, y1o, x2o, y2o = (b[:, 0:1], b[:, 1:2], b[:, 2:3], b[:, 3:4])
    ux1 = jnp.minimum(x1h, x1o)
    uy1 = jnp.minimum(y1h, y1o)
    ux2 = jnp.maximum(x2h, x2o)
    uy2 = jnp.maximum(y2h, y2o)
    i_pt = (g8 // 2).astype(jnp.float32)
    j0 = (2 * (g8 % 2)).astype(jnp.float32)
    fy = (i_pt + 0.5) / POOL
    fx0 = (j0 + 0.5) / POOL
    fx1 = (j0 + 1.5) / POOL
    yi = jnp.clip(jnp.round(uy1 + (uy2 - uy1) * fy), 0.0, FH - 1)
    xi0 = jnp.clip(jnp.round(ux1 + (ux2 - ux1) * fx0), 0.0, FW - 1)
    xi1 = jnp.clip(jnp.round(ux1 + (ux2 - ux1) * fx1), 0.0, FW - 1)
    idx = jnp.concatenate([yi * FW + xi0, yi * FW + xi1], axis=1)
    out_ref[...] = idx.astype(jnp.int32)


def _pair_indices(boxes):
    return pl.pallas_call(
        _idx_body,
        grid=(NPTS // 2, N_HUM),
        in_specs=[pl.BlockSpec((N_DET, 4), lambda g8, h: (0, 0))],
        out_specs=pl.BlockSpec((N_DET, 2), lambda g8, h: (g8 * N_HUM + h, 0)),
        out_shape=jax.ShapeDtypeStruct((P * 8, 2), jnp.int32),
    )(boxes)


def _sc_gather(featT, idx_flat):
    mesh = plsc.VectorSubcoreMesh(core_axis_name="c", subcore_axis_name="s")

    @functools.partial(
        pl.kernel,
        out_type=jax.ShapeDtypeStruct((NIDX, C), jnp.float32),
        mesh=mesh,
        compiler_params=pltpu.CompilerParams(use_tc_tiling_on_sc=False),
    )
    def gk(x_hbm, i_hbm, o_hbm):
        def body(i_vmem, o_vmem):
            pltpu.sync_copy(x_hbm.at[i_vmem.at[0]], o_vmem)

        nchunks = NIDX // GATHER_WIN
        pltpu.emit_pipeline(
            body,
            grid=(2, nchunks // 2),
            in_specs=[
                pl.BlockSpec(
                    (1, GATHER_WIN),
                    lambda i, j: (0, i * (nchunks // 2) + j),
                )
            ],
            out_specs=[
                pl.BlockSpec(
                    (GATHER_WIN, C),
                    lambda i, j: (i * (nchunks // 2) + j, 0),
                )
            ],
            core_axis_name=("c", "s"),
            dimension_semantics=(pltpu.PARALLEL, pltpu.PARALLEL),
        )(i_hbm, o_hbm)

    return gk(featT, idx_flat)


def _head_body(x0, x1, x2, x3, x4, x5, x6, x7, sc_ref, lab_ref, o2t_ref,
               w1_ref, b1_ref, w2_ref, b2_ref, w3_ref, b3_ref, out_ref):
    h = pl.program_id(0)
    rowi = jax.lax.broadcasted_iota(jnp.int32, (N_DET, 1), 0)
    hm = rowi == h
    s = sc_ref[...]
    se = jnp.where(s >= THRESH, s, 0.0)
    sh = jnp.sum(jnp.where(hm, se, 0.0))
    ds = sh * se * jnp.where(hm, 0.0, 1.0)
    lab = lab_ref[...]
    l_iota = jax.lax.broadcasted_iota(jnp.int32, (N_DET, NUM_OBJ), 1).astype(
        jnp.float32)
    ohl = jnp.where(lab == l_iota, 1.0, 0.0)
    tgt = jnp.sum(ohl * o2t_ref[...], axis=1, keepdims=True)

    x = jnp.concatenate(
        [r[...] for r in (x0, x1, x2, x3, x4, x5, x6, x7)], axis=1
    )
    h1 = jax.nn.relu(
        jnp.dot(x, w1_ref[...], preferred_element_type=jnp.float32)
        + b1_ref[...]
    )
    h2 = jax.nn.relu(
        jnp.dot(h1, w2_ref[...], preferred_element_type=jnp.float32)
        + b2_ref[...]
    )
    logits = (
        jnp.dot(h2, w3_ref[...], preferred_element_type=jnp.float32)
        + b3_ref[...]
    )
    k_iota = jax.lax.broadcasted_iota(jnp.int32, (N_DET, NUM_CLASSES), 1
                                      ).astype(jnp.float32)
    onehot = jnp.where(tgt == k_iota, 1.0, 0.0)
    out_ref[...] = ds * onehot * jax.nn.sigmoid(logits)


def _head(x2d, scores_c, labels_f, o2t_f, W1p, b1r, W2, b2r, W3, b3r):
    full = lambda shape: pl.BlockSpec(shape, lambda h: (0, 0))
    x_spec = lambda g8: pl.BlockSpec(
        (N_DET, 2 * C), lambda h, g8=g8: (g8 * N_HUM + h, 0)
    )
    return pl.pallas_call(
        _head_body,
        grid=(N_HUM,),
        in_specs=[
            x_spec(0), x_spec(1), x_spec(2), x_spec(3),
            x_spec(4), x_spec(5), x_spec(6), x_spec(7),
            full((N_DET, 1)),
            full((N_DET, 1)),
            full((1, NUM_OBJ)),
            full((C * NPTS, REP)),
            full((1, REP)),
            full((REP, REP)),
            full((1, REP)),
            full((REP, NUM_CLASSES)),
            full((1, NUM_CLASSES)),
        ],
        out_specs=pl.BlockSpec((N_DET, NUM_CLASSES), lambda h: (h, 0)),
        out_shape=jax.ShapeDtypeStruct((P, NUM_CLASSES), jnp.float32),
    )(x2d, x2d, x2d, x2d, x2d, x2d, x2d, x2d,
      scores_c, labels_f, o2t_f, W1p, b1r, W2, b2r, W3, b3r)


def kernel(features, boxes, scores, labels, W1, b1, W2, b2, W3, b3, obj2target):
    featT = features.transpose(1, 2, 0).reshape(FH * FW, C)
    W1p = (W1.reshape(C, NPTS // 2, 2, REP).transpose(1, 2, 0, 3)
           .reshape(C * NPTS, REP))
    idx = _pair_indices(boxes)
    pooled = _sc_gather(featT, idx.reshape(1, NIDX))
    x2d = pooled.reshape(P * 8, 2 * C)
    return _head(
        x2d,
        scores.reshape(N_DET, 1),
        labels.astype(jnp.float32).reshape(N_DET, 1),
        obj2target.astype(jnp.float32).reshape(1, NUM_OBJ),
        W1p,
        b1.reshape(1, REP),
        W2,
        b2.reshape(1, REP),
        W3,
        b3.reshape(1, NUM_CLASSES),
    )


# speedup vs baseline: 1.4407x; 1.4407x over previous
import functools

import jax
import jax.numpy as jnp
from jax.experimental import pallas as pl
from jax.experimental.pallas import tpu as pltpu
from jax.experimental.pallas import tpu_sc as plsc

NUM_CLASSES = 117
NUM_OBJ = 80
N_DET = 1000
N_HUM = 16
POOL = 4
NPTS = POOL * POOL
C = 64
FH = FW = 64
THRESH = 0.2
P = N_HUM * N_DET
NIDX = P * NPTS
REP = 128
GATHER_WIN = 256


def _idx_body(boxes_ref, out_ref):
    h = pl.program_id(0)
    b = boxes_ref[...]
    rowi = jax.lax.broadcasted_iota(jnp.int32, (N_DET, 1), 0)
    hm = rowi == h
    bh = jnp.sum(jnp.where(hm, b, 0.0), axis=0, keepdims=True)
    x1h, y1h, x2h, y2h = (bh[:, 0:1], bh[:, 1:2], bh[:, 2:3], bh[:, 3:4])
    x1o, y1o, x2o, y2o = (b[:, 0:1], b[:, 1:2], b[:, 2:3], b[:, 3:4])
    ux1 = jnp.minimum(x1h, x1o)
    uy1 = jnp.minimum(y1h, y1o)
    ux2 = jnp.maximum(x2h, x2o)
    uy2 = jnp.maximum(y2h, y2o)
    chunks = []
    for g8 in range(NPTS // 2):
        i_pt, j0 = g8 // 2, 2 * (g8 % 2)
        fy = (i_pt + 0.5) / POOL
        fx0 = (j0 + 0.5) / POOL
        fx1 = (j0 + 1.5) / POOL
        yi = jnp.clip(jnp.round(uy1 + (uy2 - uy1) * fy), 0.0, FH - 1)
        xi0 = jnp.clip(jnp.round(ux1 + (ux2 - ux1) * fx0), 0.0, FW - 1)
        xi1 = jnp.clip(jnp.round(ux1 + (ux2 - ux1) * fx1), 0.0, FW - 1)
        chunks.append(jnp.concatenate([yi * FW + xi0, yi * FW + xi1], axis=1))
    out_ref[...] = jnp.concatenate(chunks, axis=0).astype(jnp.int32)


def _pair_indices(boxes):
    return pl.pallas_call(
        _idx_body,
        grid=(N_HUM,),
        in_specs=[pl.BlockSpec((N_DET, 4), lambda h: (0, 0))],
        out_specs=pl.BlockSpec((8 * N_DET, 2), lambda h: (h, 0)),
        out_shape=jax.ShapeDtypeStruct((P * 8, 2), jnp.int32),
    )(boxes)


def _sc_gather(featT, idx_flat):
    mesh = plsc.VectorSubcoreMesh(core_axis_name="c", subcore_axis_name="s")

    @functools.partial(
        pl.kernel,
        out_type=jax.ShapeDtypeStruct((NIDX, C), jnp.float32),
        mesh=mesh,
        compiler_params=pltpu.CompilerParams(use_tc_tiling_on_sc=False),
    )
    def gk(x_hbm, i_hbm, o_hbm):
        def body(i_vmem, o_vmem):
            pltpu.sync_copy(x_hbm.at[i_vmem.at[0]], o_vmem)

        nchunks = NIDX // GATHER_WIN
        pltpu.emit_pipeline(
            body,
            grid=(2, nchunks // 2),
            in_specs=[
                pl.BlockSpec(
                    (1, GATHER_WIN),
                    lambda i, j: (0, i * (nchunks // 2) + j),
                )
            ],
            out_specs=[
                pl.BlockSpec(
                    (GATHER_WIN, C),
                    lambda i, j: (i * (nchunks // 2) + j, 0),
                )
            ],
            core_axis_name=("c", "s"),
            dimension_semantics=(pltpu.PARALLEL, pltpu.PARALLEL),
        )(i_hbm, o_hbm)

    return gk(featT, idx_flat)


def _head_body(x_ref, sc_ref, lab_ref, o2t_ref,
               w1_ref, b1_ref, w2_ref, b2_ref, w3_ref, b3_ref, out_ref):
    h = pl.program_id(0)
    rowi = jax.lax.broadcasted_iota(jnp.int32, (N_DET, 1), 0)
    hm = rowi == h
    s = sc_ref[...]
    se = jnp.where(s >= THRESH, s, 0.0)
    sh = jnp.sum(jnp.where(hm, se, 0.0))
    ds = sh * se * jnp.where(hm, 0.0, 1.0)
    lab = lab_ref[...]
    l_iota = jax.lax.broadcasted_iota(jnp.int32, (N_DET, NUM_OBJ), 1).astype(
        jnp.float32)
    ohl = jnp.where(lab == l_iota, 1.0, 0.0)
    tgt = jnp.sum(ohl * o2t_ref[...], axis=1, keepdims=True)

    x8 = x_ref[...]
    x = jnp.concatenate(
        [x8[g8 * N_DET:(g8 + 1) * N_DET, :] for g8 in range(NPTS // 2)],
        axis=1,
    )
    h1 = jax.nn.relu(
        jnp.dot(x, w1_ref[...], preferred_element_type=jnp.float32)
        + b1_ref[...]
    )
    h2 = jax.nn.relu(
        jnp.dot(h1, w2_ref[...], preferred_element_type=jnp.float32)
        + b2_ref[...]
    )
    logits = (
        jnp.dot(h2, w3_ref[...], preferred_element_type=jnp.float32)
        + b3_ref[...]
    )
    k_iota = jax.lax.broadcasted_iota(jnp.int32, (N_DET, NUM_CLASSES), 1
                                      ).astype(jnp.float32)
    onehot = jnp.where(tgt == k_iota, 1.0, 0.0)
    out_ref[...] = ds * onehot * jax.nn.sigmoid(logits)


def _head(x2d, scores_c, labels_f, o2t_f, W1p, b1r, W2, b2r, W3, b3r):
    full = lambda shape: pl.BlockSpec(shape, lambda h: (0, 0))
    return pl.pallas_call(
        _head_body,
        grid=(N_HUM,),
        in_specs=[
            pl.BlockSpec((8 * N_DET, 2 * C), lambda h: (h, 0)),
            full((N_DET, 1)),
            full((N_DET, 1)),
            full((1, NUM_OBJ)),
            full((C * NPTS, REP)),
            full((1, REP)),
            full((REP, REP)),
            full((1, REP)),
            full((REP, NUM_CLASSES)),
            full((1, NUM_CLASSES)),
        ],
        out_specs=pl.BlockSpec((N_DET, NUM_CLASSES), lambda h: (h, 0)),
        out_shape=jax.ShapeDtypeStruct((P, NUM_CLASSES), jnp.float32),
    )(x2d, scores_c, labels_f, o2t_f, W1p, b1r, W2, b2r, W3, b3r)


def kernel(features, boxes, scores, labels, W1, b1, W2, b2, W3, b3, obj2target):
    featT = features.transpose(1, 2, 0).reshape(FH * FW, C)
    W1p = (W1.reshape(C, NPTS // 2, 2, REP).transpose(1, 2, 0, 3)
           .reshape(C * NPTS, REP))
    idx = _pair_indices(boxes)
    pooled = _sc_gather(featT, idx.reshape(1, NIDX))
    x2d = pooled.reshape(P * 8, 2 * C)
    return _head(
        x2d,
        scores.reshape(N_DET, 1),
        labels.astype(jnp.float32).reshape(N_DET, 1),
        obj2target.astype(jnp.float32).reshape(1, NUM_OBJ),
        W1p,
        b1.reshape(1, REP),
        W2,
        b2.reshape(1, REP),
        W3,
        b3.reshape(1, NUM_CLASSES),
    )
